# Initial kernel scaffold; baseline (speedup 1.0000x reference)
import jax
import jax.numpy as jnp
from jax.experimental import pallas as pl


def kernel(word, feat, W_word, W_feat):
    raise NotImplementedError("write your pallas kernel here")


# speedup vs baseline: 2.5678x; 2.5678x over previous
import functools

import jax
import jax.numpy as jnp
from jax import lax
from jax.experimental import pallas as pl
from jax.experimental.pallas import tpu as pltpu
from jax.experimental.pallas import tpu_sc as plsc

_B, _L = 4096, 50
_DW, _DF = 128, 64
_DO = _DW + _DF
_N = _B * _L
_NW = 32
_RPW = _N // _NW
_CHUNK = 128
_NCHUNK = _RPW // _CHUNK
_CWORDS = _CHUNK * _DO

_mesh = plsc.VectorSubcoreMesh(core_axis_name="c", subcore_axis_name="s")


@functools.partial(
    pl.kernel,
    out_type=jax.ShapeDtypeStruct((_N * _DO,), jnp.float32),
    mesh=_mesh,
    scratch_types=[
        pltpu.VMEM((_NCHUNK, _CHUNK), jnp.int32),
        pltpu.VMEM((_NCHUNK, _CHUNK), jnp.int32),
        pltpu.VMEM((_CHUNK, _DW), jnp.float32),
        pltpu.VMEM((_CHUNK, _DW), jnp.float32),
        pltpu.VMEM((_CWORDS,), jnp.float32),
        pltpu.SemaphoreType.DMA,
        pltpu.SemaphoreType.DMA,
    ],
)
def _stack_kernel(word_hbm, feat_hbm, ww_hbm, wf_hbm, out_hbm,
                  idxw_v, idxf_v, rw_v, rf_v, asm_v, semw, semf):
    wid = lax.axis_index("s") * 2 + lax.axis_index("c")
    pltpu.sync_copy(word_hbm.at[wid], idxw_v)
    pltpu.sync_copy(feat_hbm.at[wid], idxf_v)

    @pl.loop(0, _NCHUNK)
    def _chunk(j):
        cw = pltpu.async_copy(ww_hbm.at[idxw_v.at[j]], rw_v, semw)
        cf = pltpu.async_copy(wf_hbm.at[idxf_v.at[j]], rf_v, semf)
        cw.wait()
        cf.wait()

        @pl.loop(0, _CHUNK)
        def _row(r):
            off = r * _DO
            for k in range(_DW // 16):
                asm_v[pl.ds(off + 16 * k, 16)] = rw_v[r, pl.ds(16 * k, 16)]
            for k in range(_DF // 16):
                asm_v[pl.ds(off + _DW + 16 * k, 16)] = rf_v[r, pl.ds(16 * k, 16)]

        flat = (wid * _RPW + j * _CHUNK) * _DO
        pltpu.sync_copy(asm_v, out_hbm.at[pl.ds(flat, _CWORDS)])


def kernel(word, feat, W_word, W_feat):
    word3 = word.reshape(_NW, _NCHUNK, _CHUNK).astype(jnp.int32)
    feat3 = feat.reshape(_NW, _NCHUNK, _CHUNK).astype(jnp.int32)
    wf_pad = jnp.pad(W_feat, ((0, 0), (0, _DW - _DF)))
    out = _stack_kernel(word3, feat3, W_word, wf_pad)
    return out.reshape(_B, _L, _DO)


# speedup vs baseline: 3.1528x; 1.2278x over previous
import functools

import jax
import jax.numpy as jnp
from jax import lax
from jax.experimental import pallas as pl
from jax.experimental.pallas import tpu as pltpu
from jax.experimental.pallas import tpu_sc as plsc

_B, _L = 4096, 50
_DW, _DF = 128, 64
_DO = _DW + _DF
_N = _B * _L
_NW = 32
_RPW = _N // _NW
_CHUNK = 128
_NCHUNK = _RPW // _CHUNK
_CWORDS = _CHUNK * _DO

_mesh = plsc.VectorSubcoreMesh(core_axis_name="c", subcore_axis_name="s")


@functools.partial(
    pl.kernel,
    out_type=jax.ShapeDtypeStruct((_N * _DO,), jnp.float32),
    mesh=_mesh,
    scratch_types=[
        pltpu.VMEM((_NCHUNK, _CHUNK), jnp.int32),
        pltpu.VMEM((_NCHUNK, _CHUNK), jnp.int32),
        [pltpu.VMEM((_CHUNK, _DW), jnp.float32)] * 2,
        [pltpu.VMEM((_CHUNK, _DW), jnp.float32)] * 2,
        [pltpu.VMEM((_CWORDS,), jnp.float32)] * 2,
        [pltpu.SemaphoreType.DMA] * 2,
        [pltpu.SemaphoreType.DMA] * 2,
        [pltpu.SemaphoreType.DMA] * 2,
    ],
)
def _stack_kernel(word_hbm, feat_hbm, ww_hbm, wf_hbm, out_hbm,
                  idxw_v, idxf_v, rw, rf, asm, semw, semf, semo):
    wid = lax.axis_index("s") * 2 + lax.axis_index("c")
    pltpu.sync_copy(word_hbm.at[wid], idxw_v)
    pltpu.sync_copy(feat_hbm.at[wid], idxf_v)

    def fire(c, b):
        pltpu.async_copy(ww_hbm.at[idxw_v.at[c]], rw[b], semw[b])
        pltpu.async_copy(wf_hbm.at[idxf_v.at[c]], rf[b], semf[b])

    def process(c, b, first):
        pltpu.make_async_copy(ww_hbm.at[idxw_v.at[c]], rw[b], semw[b]).wait()
        pltpu.make_async_copy(wf_hbm.at[idxf_v.at[c]], rf[b], semf[b]).wait()
        @pl.when(jnp.logical_not(first))
        def _():
            pltpu.make_async_copy(
                asm[b], out_hbm.at[pl.ds(0, _CWORDS)], semo[b]).wait()

        @pl.loop(0, _CHUNK)
        def _row(r):
            off = r * _DO
            for k in range(_DW // 16):
                asm[b][pl.ds(off + 16 * k, 16)] = rw[b][r, pl.ds(16 * k, 16)]
            for k in range(_DF // 16):
                asm[b][pl.ds(off + _DW + 16 * k, 16)] = rf[b][r, pl.ds(16 * k, 16)]

        flat = (wid * _RPW + c * _CHUNK) * _DO
        pltpu.async_copy(asm[b], out_hbm.at[pl.ds(flat, _CWORDS)], semo[b])

    fire(0, 0)

    @pl.loop(0, _NCHUNK // 2)
    def _pair(p):
        c0 = 2 * p
        fire(c0 + 1, 1)
        process(c0, 0, p == 0)

        @pl.when(p < _NCHUNK // 2 - 1)
        def _():
            fire(c0 + 2, 0)
        process(c0 + 1, 1, p == 0)

    pltpu.make_async_copy(asm[0], out_hbm.at[pl.ds(0, _CWORDS)], semo[0]).wait()
    pltpu.make_async_copy(asm[1], out_hbm.at[pl.ds(0, _CWORDS)], semo[1]).wait()


def kernel(word, feat, W_word, W_feat):
    word3 = word.reshape(_NW, _NCHUNK, _CHUNK).astype(jnp.int32)
    feat3 = feat.reshape(_NW, _NCHUNK, _CHUNK).astype(jnp.int32)
    wf_pad = jnp.pad(W_feat, ((0, 0), (0, _DW - _DF)))
    out = _stack_kernel(word3, feat3, W_word, wf_pad)
    return out.reshape(_B, _L, _DO)


# speedup vs baseline: 4.1487x; 1.3159x over previous
import functools

import jax
import jax.numpy as jnp
from jax import lax
from jax.experimental import pallas as pl
from jax.experimental.pallas import tpu as pltpu
from jax.experimental.pallas import tpu_sc as plsc

_B, _L = 4096, 50
_DW, _DF = 128, 64
_DO = _DW + _DF
_NW = 32
_BPW = _B // _NW
_GB = 2
_GT = _GB * _L
_NG = _BPW // _GB

_mesh = plsc.VectorSubcoreMesh(core_axis_name="c", subcore_axis_name="s")


@functools.partial(
    pl.kernel,
    out_type=jax.ShapeDtypeStruct((_B, _L, _DO), jnp.float32),
    mesh=_mesh,
    scratch_types=[
        pltpu.VMEM((_NG, _GT), jnp.int32),
        pltpu.VMEM((_NG, _GT), jnp.int32),
        [pltpu.VMEM((_GT, _DW), jnp.float32)] * 2,
        [pltpu.VMEM((_GT, _DW), jnp.float32)] * 2,
        [pltpu.VMEM((_GB, _L, _DO), jnp.float32)] * 2,
        [pltpu.SemaphoreType.DMA] * 2,
        [pltpu.SemaphoreType.DMA] * 2,
        [pltpu.SemaphoreType.DMA] * 2,
    ],
)
def _stack_kernel(word_hbm, feat_hbm, ww_hbm, wf_hbm, out_hbm,
                  idxw_v, idxf_v, rw, rf, asm, semw, semf, semo):
    wid = lax.axis_index("s") * 2 + lax.axis_index("c")
    pltpu.sync_copy(word_hbm.at[wid], idxw_v)
    pltpu.sync_copy(feat_hbm.at[wid], idxf_v)

    def fire(c, b):
        pltpu.async_copy(ww_hbm.at[idxw_v.at[c]], rw[b], semw[b])
        pltpu.async_copy(wf_hbm.at[idxf_v.at[c]], rf[b], semf[b])

    def process(c, b, first):
        pltpu.make_async_copy(ww_hbm.at[idxw_v.at[c]], rw[b], semw[b]).wait()
        pltpu.make_async_copy(wf_hbm.at[idxf_v.at[c]], rf[b], semf[b]).wait()
        @pl.when(jnp.logical_not(first))
        def _():
            pltpu.make_async_copy(
                asm[b], out_hbm.at[pl.ds(0, _GB)], semo[b]).wait()

        for g in range(_GB):
            @pl.loop(0, _L)
            def _row(l):
                r = g * _L + l
                for k in range(_DW // 16):
                    asm[b][g, l, pl.ds(16 * k, 16)] = rw[b][r, pl.ds(16 * k, 16)]
                for k in range(_DF // 16):
                    asm[b][g, l, pl.ds(_DW + 16 * k, 16)] = rf[b][r, pl.ds(16 * k, 16)]

        b0 = wid * _BPW + c * _GB
        pltpu.async_copy(asm[b], out_hbm.at[pl.ds(b0, _GB)], semo[b])

    fire(0, 0)

    @pl.loop(0, _NG // 2)
    def _pair(p):
        c0 = 2 * p
        fire(c0 + 1, 1)
        process(c0, 0, p == 0)

        @pl.when(p < _NG // 2 - 1)
        def _():
            fire(c0 + 2, 0)
        process(c0 + 1, 1, p == 0)

    pltpu.make_async_copy(asm[0], out_hbm.at[pl.ds(0, _GB)], semo[0]).wait()
    pltpu.make_async_copy(asm[1], out_hbm.at[pl.ds(0, _GB)], semo[1]).wait()


def kernel(word, feat, W_word, W_feat):
    word3 = word.reshape(_NW, _NG, _GT).astype(jnp.int32)
    feat3 = feat.reshape(_NW, _NG, _GT).astype(jnp.int32)
    wf_pad = jnp.pad(W_feat, ((0, 0), (0, _DW - _DF)))
    return _stack_kernel(word3, feat3, W_word, wf_pad)
